# baseline (device time: 181224 ns/iter reference)
import jax
import jax.numpy as jnp
from jax import lax
from jax.experimental import pallas as pl
from jax.experimental.pallas import tpu as pltpu

BLKV = 1024


def kernel(x, W, labels):
    T, D = x.shape
    _, V_shard = W.shape
    Th = T // 2
    nsteps = V_shard // BLKV

    my_x = lax.axis_index("x")
    my_y = lax.axis_index("y")

    xh32 = lax.dynamic_slice_in_dim(x, my_x * Th, Th, 0)
    labh = lax.dynamic_slice_in_dim(labels, my_x * Th, Th)
    loc = labh - my_y * V_shard
    in_shard = (loc >= 0) & (loc < V_shard)
    Wg = jnp.take(W, jnp.clip(loc, 0, V_shard - 1), axis=1)
    ll = jnp.where(in_shard,
                   jnp.sum(xh32.astype(jnp.bfloat16).astype(jnp.float32)
                           * Wg.T.astype(jnp.bfloat16).astype(jnp.float32),
                           axis=1),
                   0.0).reshape(Th, 1)

    def body(x_ref, w_ref, ll_ref, out_ref,
             s_ref, csend_y, crecv_y, csend_x, crecv_x,
             send_y_sem, recv_y_sem, send_x_sem, recv_x_sem):
        i = pl.program_id(0)
        my_x = lax.axis_index("x")
        my_y = lax.axis_index("y")

        @pl.when(i == 0)
        def _():
            s_ref[...] = jnp.zeros((Th, 1), jnp.float32)

        w = w_ref[...].astype(jnp.bfloat16)
        logits = jnp.dot(x_ref[...], w, preferred_element_type=jnp.float32)
        s_ref[...] += jnp.sum(jnp.exp(logits), axis=1, keepdims=True)

        @pl.when(i == nsteps - 1)
        def _():
            barrier = pltpu.get_barrier_semaphore()
            for nbr in ((my_x, 1 - my_y), (1 - my_x, my_y)):
                pl.semaphore_signal(barrier, inc=1, device_id=nbr,
                                    device_id_type=pl.DeviceIdType.MESH)
            pl.semaphore_wait(barrier, 2)

            csend_y[:, 0:1] = s_ref[...]
            csend_y[:, 1:2] = ll_ref[...]
            rdma_y = pltpu.make_async_remote_copy(
                src_ref=csend_y, dst_ref=crecv_y,
                send_sem=send_y_sem, recv_sem=recv_y_sem,
                device_id=(my_x, 1 - my_y),
                device_id_type=pl.DeviceIdType.MESH,
            )
            rdma_y.start()
            rdma_y.wait()

            nll = (jnp.log(s_ref[...] + crecv_y[:, 0:1])
                   - (ll_ref[...] + crecv_y[:, 1:2]))

            csend_x[...] = nll
            rdma_x = pltpu.make_async_remote_copy(
                src_ref=csend_x, dst_ref=crecv_x,
                send_sem=send_x_sem, recv_sem=recv_x_sem,
                device_id=(1 - my_x, my_y),
                device_id_type=pl.DeviceIdType.MESH,
            )
            rdma_x.start()
            rdma_x.wait()

            @pl.when(my_x == 0)
            def _():
                out_ref[0:Th, :] = nll
                out_ref[Th:T, :] = crecv_x[...]

            @pl.when(my_x == 1)
            def _():
                out_ref[0:Th, :] = crecv_x[...]
                out_ref[Th:T, :] = nll

    out = pl.pallas_call(
        body,
        grid=(nsteps,),
        out_shape=jax.ShapeDtypeStruct((T, 1), jnp.float32),
        in_specs=[
            pl.BlockSpec((Th, D), lambda i: (0, 0)),
            pl.BlockSpec((D, BLKV), lambda i: (0, i)),
            pl.BlockSpec((Th, 1), lambda i: (0, 0)),
        ],
        out_specs=pl.BlockSpec((T, 1), lambda i: (0, 0)),
        scratch_shapes=[
            pltpu.VMEM((Th, 1), jnp.float32),
            pltpu.VMEM((Th, 2), jnp.float32),
            pltpu.VMEM((Th, 2), jnp.float32),
            pltpu.VMEM((Th, 1), jnp.float32),
            pltpu.VMEM((Th, 1), jnp.float32),
            pltpu.SemaphoreType.DMA,
            pltpu.SemaphoreType.DMA,
            pltpu.SemaphoreType.DMA,
            pltpu.SemaphoreType.DMA,
        ],
        compiler_params=pltpu.CompilerParams(
            collective_id=0,
            dimension_semantics=("arbitrary",),
        ),
    )(xh32.astype(jnp.bfloat16), W, ll)
    return out.reshape(T)


# device time: 67442 ns/iter; 2.6871x vs baseline; 2.6871x over previous
import jax
import jax.numpy as jnp
from jax import lax
from jax.experimental import pallas as pl
from jax.experimental.pallas import tpu as pltpu

BLKV = 1024


def kernel(x, W, labels):
    T, D = x.shape
    _, V_shard = W.shape
    Th = T // 2
    nsteps = V_shard // BLKV

    my_x0 = lax.axis_index("x")
    xh = lax.dynamic_slice_in_dim(x, my_x0 * Th, Th, 0).astype(jnp.bfloat16)
    labh = lax.dynamic_slice_in_dim(labels, my_x0 * Th, Th).reshape(Th, 1)

    def body(x_ref, w_ref, lab_ref, out_ref,
             s_ref, ll_ref, csend_y, crecv_y, csend_x, crecv_x,
             send_y_sem, recv_y_sem, send_x_sem, recv_x_sem):
        i = pl.program_id(0)
        my_x = lax.axis_index("x")
        my_y = lax.axis_index("y")

        @pl.when(i == 0)
        def _():
            s_ref[...] = jnp.zeros((Th, 1), jnp.float32)
            ll_ref[...] = jnp.zeros((Th, 1), jnp.float32)

        w = w_ref[...].astype(jnp.bfloat16)
        logits = jnp.dot(x_ref[...], w, preferred_element_type=jnp.float32)
        s_ref[...] += jnp.sum(jnp.exp(logits), axis=1, keepdims=True)

        local = lab_ref[...] - (my_y * V_shard + i * BLKV)
        cols = lax.broadcasted_iota(jnp.int32, (Th, BLKV), 1)
        ll_ref[...] += jnp.sum(jnp.where(cols == local, logits, 0.0),
                               axis=1, keepdims=True)

        @pl.when(i == nsteps - 1)
        def _():
            barrier = pltpu.get_barrier_semaphore()
            for nbr in ((my_x, 1 - my_y), (1 - my_x, my_y)):
                pl.semaphore_signal(barrier, inc=1, device_id=nbr,
                                    device_id_type=pl.DeviceIdType.MESH)
            pl.semaphore_wait(barrier, 2)

            csend_y[:, 0:1] = s_ref[...]
            csend_y[:, 1:2] = ll_ref[...]
            rdma_y = pltpu.make_async_remote_copy(
                src_ref=csend_y, dst_ref=crecv_y,
                send_sem=send_y_sem, recv_sem=recv_y_sem,
                device_id=(my_x, 1 - my_y),
                device_id_type=pl.DeviceIdType.MESH,
            )
            rdma_y.start()
            rdma_y.wait()

            nll = (jnp.log(s_ref[...] + crecv_y[:, 0:1])
                   - (ll_ref[...] + crecv_y[:, 1:2]))

            csend_x[...] = nll
            rdma_x = pltpu.make_async_remote_copy(
                src_ref=csend_x, dst_ref=crecv_x,
                send_sem=send_x_sem, recv_sem=recv_x_sem,
                device_id=(1 - my_x, my_y),
                device_id_type=pl.DeviceIdType.MESH,
            )
            rdma_x.start()
            rdma_x.wait()

            @pl.when(my_x == 0)
            def _():
                out_ref[0:Th, :] = nll
                out_ref[Th:T, :] = crecv_x[...]

            @pl.when(my_x == 1)
            def _():
                out_ref[0:Th, :] = crecv_x[...]
                out_ref[Th:T, :] = nll

    out = pl.pallas_call(
        body,
        grid=(nsteps,),
        out_shape=jax.ShapeDtypeStruct((T, 1), jnp.float32),
        in_specs=[
            pl.BlockSpec((Th, D), lambda i: (0, 0)),
            pl.BlockSpec((D, BLKV), lambda i: (0, i)),
            pl.BlockSpec((Th, 1), lambda i: (0, 0)),
        ],
        out_specs=pl.BlockSpec((T, 1), lambda i: (0, 0)),
        scratch_shapes=[
            pltpu.VMEM((Th, 1), jnp.float32),
            pltpu.VMEM((Th, 1), jnp.float32),
            pltpu.VMEM((Th, 2), jnp.float32),
            pltpu.VMEM((Th, 2), jnp.float32),
            pltpu.VMEM((Th, 1), jnp.float32),
            pltpu.VMEM((Th, 1), jnp.float32),
            pltpu.SemaphoreType.DMA,
            pltpu.SemaphoreType.DMA,
            pltpu.SemaphoreType.DMA,
            pltpu.SemaphoreType.DMA,
        ],
        compiler_params=pltpu.CompilerParams(
            collective_id=0,
            dimension_semantics=("arbitrary",),
        ),
    )(xh, W, labh)
    return out.reshape(T)
